# baseline (speedup 1.0000x reference)
import functools

import jax
import jax.numpy as jnp
from jax import lax
from jax.experimental import pallas as pl
from jax.experimental.pallas import tpu as pltpu
from jax.experimental.pallas import tpu_sc as plsc

N_NODES = 10000
N_EDGES = 320000
D_FEAT = 128
D_EDGE = 16
D_OUT = 128

N_HALF = N_EDGES // 2

NC = 2
NS = 16
NW = NC * NS
PER_W = N_HALF // NW
CHUNK = 40
N_CHUNKS = PER_W // CHUNK

_BE = 20000
_N_EBLK = N_HALF // _BE


def _tc_a_body(ea_ref, wep_ref, x_ref, ws_ref, wr_ref, b_ref,
               ep_ref, ts_ref, tr_ref):
    ep_ref[...] = jnp.dot(ea_ref[...], wep_ref[...], preferred_element_type=jnp.float32)

    @pl.when(pl.program_id(0) == 0)
    def _():
        xb = x_ref[...]
        ts_ref[...] = jnp.dot(xb, ws_ref[...], preferred_element_type=jnp.float32) + b_ref[...]
        tr_ref[...] = jnp.dot(xb, wr_ref[...], preferred_element_type=jnp.float32)


def _tc_a(ea, wep, x, ws, wr, b2d):
    nidx = lambda i: (0, 0)
    return pl.pallas_call(
        _tc_a_body,
        grid=(_N_EBLK,),
        in_specs=[
            pl.BlockSpec((_BE, D_EDGE), lambda i: (i, 0)),
            pl.BlockSpec((D_EDGE, D_OUT), lambda i: (0, 0)),
            pl.BlockSpec((N_NODES, D_FEAT), nidx),
            pl.BlockSpec((D_FEAT, D_OUT), lambda i: (0, 0)),
            pl.BlockSpec((D_FEAT, D_OUT), lambda i: (0, 0)),
            pl.BlockSpec((1, D_OUT), lambda i: (0, 0)),
        ],
        out_specs=[
            pl.BlockSpec((_BE, D_OUT), lambda i: (i, 0)),
            pl.BlockSpec((N_NODES, D_OUT), nidx),
            pl.BlockSpec((N_NODES, D_OUT), nidx),
        ],
        out_shape=[
            jax.ShapeDtypeStruct((N_HALF, D_OUT), jnp.float32),
            jax.ShapeDtypeStruct((N_NODES, D_OUT), jnp.float32),
            jax.ShapeDtypeStruct((N_NODES, D_OUT), jnp.float32),
        ],
    )(ea, wep, x, ws, wr, b2d)


def _tc_b_body(ea_ref, wep_ref, ep_ref):
    ep_ref[...] = jnp.dot(ea_ref[...], wep_ref[...], preferred_element_type=jnp.float32)


def _tc_b(ea, wep):
    return pl.pallas_call(
        _tc_b_body,
        grid=(_N_EBLK,),
        in_specs=[
            pl.BlockSpec((_BE, D_EDGE), lambda i: (i, 0)),
            pl.BlockSpec((D_EDGE, D_OUT), lambda i: (0, 0)),
        ],
        out_specs=pl.BlockSpec((_BE, D_OUT), lambda i: (i, 0)),
        out_shape=jax.ShapeDtypeStruct((N_HALF, D_OUT), jnp.float32),
    )(ea, wep)


_MESH = plsc.VectorSubcoreMesh(
    core_axis_name="c", subcore_axis_name="s", num_cores=NC, num_subcores=NS
)


@functools.partial(
    pl.kernel,
    mesh=_MESH,
    out_type=jax.ShapeDtypeStruct((N_HALF, D_OUT), jnp.float32),
    scratch_types=[
        pltpu.VMEM((2, 2, CHUNK), jnp.int32),
        pltpu.VMEM((2, CHUNK, D_OUT), jnp.float32),
        pltpu.VMEM((2, CHUNK, D_OUT), jnp.float32),
        pltpu.VMEM((2, CHUNK, D_OUT), jnp.float32),
        [pltpu.SemaphoreType.DMA] * 2,
        [pltpu.SemaphoreType.DMA] * 2,
        [pltpu.SemaphoreType.DMA] * 2,
        [pltpu.SemaphoreType.DMA] * 2,
        [pltpu.SemaphoreType.DMA] * 2,
    ],
)
def _sc_gather_sum(ts_hbm, tr_hbm, ep_hbm, sidx_hbm, ridx_hbm, out_hbm,
                   idx_v, a_v, b_v, o_v, si, sa, sb, sc, so):
    wid = lax.axis_index("s") * NC + lax.axis_index("c")
    base = wid * PER_W

    def issue_gathers(ci, k):
        off = base + ci * CHUNK
        pltpu.async_copy(ts_hbm.at[idx_v.at[k, 0]], a_v.at[k], sa[k])
        pltpu.async_copy(tr_hbm.at[idx_v.at[k, 1]], b_v.at[k], sb[k])
        pltpu.async_copy(ep_hbm.at[pl.ds(off, CHUNK)], o_v.at[k], sc[k])

    def wait_gathers(k):
        pltpu.make_async_copy(ts_hbm.at[idx_v.at[k, 0]], a_v.at[k], sa[k]).wait()
        pltpu.make_async_copy(tr_hbm.at[idx_v.at[k, 1]], b_v.at[k], sb[k]).wait()
        pltpu.make_async_copy(ep_hbm.at[pl.ds(0, CHUNK)], o_v.at[k], sc[k]).wait()

    def prefetch_idx(ci, k):
        off = base + ci * CHUNK
        pltpu.async_copy(sidx_hbm.at[pl.ds(off, CHUNK)], idx_v.at[k, 0], si[k])
        pltpu.async_copy(ridx_hbm.at[pl.ds(off, CHUNK)], idx_v.at[k, 1], si[k])

    def wait_idx(k):
        pltpu.make_async_copy(sidx_hbm.at[pl.ds(0, CHUNK)], idx_v.at[k, 0], si[k]).wait()
        pltpu.make_async_copy(ridx_hbm.at[pl.ds(0, CHUNK)], idx_v.at[k, 1], si[k]).wait()

    def drain_out(k):
        pltpu.make_async_copy(o_v.at[k], out_hbm.at[pl.ds(0, CHUNK)], so[k]).wait()

    pltpu.sync_copy(sidx_hbm.at[pl.ds(base, CHUNK)], idx_v.at[0, 0])
    pltpu.sync_copy(ridx_hbm.at[pl.ds(base, CHUNK)], idx_v.at[0, 1])
    issue_gathers(0, 0)
    prefetch_idx(1, 1)

    def chunk_body(ci, k):
            nk = 1 - k

            @pl.when(ci + 1 < N_CHUNKS)
            def _():
                wait_idx(nk)

                @pl.when(ci >= 1)
                def _():
                    drain_out(nk)

                issue_gathers(ci + 1, nk)

            wait_gathers(k)

            @pl.when(ci + 2 < N_CHUNKS)
            def _():
                prefetch_idx(ci + 2, k)

            @plsc.parallel_loop(0, CHUNK, unroll=2)
            def _(e):
                for j in range(D_OUT // 16):
                    sl = pl.ds(j * 16, 16)
                    plsc.addupdate(o_v.at[k, e, sl], a_v[k, e, sl] + b_v[k, e, sl])
            off = base + ci * CHUNK
            pltpu.async_copy(o_v.at[k], out_hbm.at[pl.ds(off, CHUNK)], so[k])

    def pair_body(g, carry):
        chunk_body(g, 0)
        chunk_body(g + 1, 1)
        return carry

    lax.fori_loop(0, N_CHUNKS // 2, lambda g, c: pair_body(g * 2, c), 0)
    if N_CHUNKS % 2:
        chunk_body(N_CHUNKS - 1, 0)
    drain_out((N_CHUNKS - 2) % 2)
    drain_out((N_CHUNKS - 1) % 2)


def kernel(x, edge_index, edge_attr, coords, W, b):
    ws = W[:D_FEAT]
    wr = W[D_FEAT:2 * D_FEAT]
    wep = W[2 * D_FEAT:]
    b2d = b.reshape(1, D_OUT)

    sidx = edge_index[0].astype(jnp.int32)
    ridx = edge_index[1].astype(jnp.int32)

    ep_a, ts, tr = _tc_a(edge_attr[:N_HALF], wep, x, ws, wr, b2d)
    out_a = _sc_gather_sum(ts, tr, ep_a, sidx[:N_HALF], ridx[:N_HALF])
    ep_b = _tc_b(edge_attr[N_HALF:], wep)
    out_b = _sc_gather_sum(ts, tr, ep_b, sidx[N_HALF:], ridx[N_HALF:])

    edge_attr_ = jnp.concatenate([out_a, out_b], axis=0)
    return (coords, x, edge_attr_, edge_index)


# speedup vs baseline: 1.2835x; 1.2835x over previous
import functools

import jax
import jax.numpy as jnp
import numpy as np
from jax import lax
from jax.experimental import pallas as pl
from jax.experimental.pallas import tpu as pltpu
from jax.experimental.pallas import tpu_sc as plsc

N_NODES = 10000
N_EDGES = 320000
D_FEAT = 128
D_EDGE = 16
D_OUT = 128

NC = 2
NS = 16
NW = NC * NS
PER_W = N_EDGES // NW
CHUNK = 80
N_CHUNKS = PER_W // CHUNK

_BE = 20000
_BN = N_NODES
_N_EBLK = N_EDGES // _BE

def _tc_body(ea_ref, wep_ref, x_ref, ws_ref, wr_ref, b_ref,
             ep_ref, ts_ref, tr_ref):
    ep_ref[...] = jnp.dot(ea_ref[...], wep_ref[...], preferred_element_type=jnp.float32)

    @pl.when(pl.program_id(0) == 0)
    def _():
        xb = x_ref[...]
        ts_ref[...] = jnp.dot(xb, ws_ref[...], preferred_element_type=jnp.float32) + b_ref[...]
        tr_ref[...] = jnp.dot(xb, wr_ref[...], preferred_element_type=jnp.float32)


def _tc_stage(ea, wep, x, ws, wr, b2d):
    nidx = lambda i: (0, 0)
    return pl.pallas_call(
        _tc_body,
        grid=(_N_EBLK,),
        in_specs=[
            pl.BlockSpec((_BE, D_EDGE), lambda i: (i, 0)),
            pl.BlockSpec((D_EDGE, D_OUT), lambda i: (0, 0)),
            pl.BlockSpec((_BN, D_FEAT), nidx),
            pl.BlockSpec((D_FEAT, D_OUT), lambda i: (0, 0)),
            pl.BlockSpec((D_FEAT, D_OUT), lambda i: (0, 0)),
            pl.BlockSpec((1, D_OUT), lambda i: (0, 0)),
        ],
        out_specs=[
            pl.BlockSpec((_BE, D_OUT), lambda i: (i, 0)),
            pl.BlockSpec((_BN, D_OUT), nidx),
            pl.BlockSpec((_BN, D_OUT), nidx),
        ],
        out_shape=[
            jax.ShapeDtypeStruct((N_EDGES, D_OUT), jnp.float32),
            jax.ShapeDtypeStruct((N_NODES, D_OUT), jnp.float32),
            jax.ShapeDtypeStruct((N_NODES, D_OUT), jnp.float32),
        ],
    )(ea, wep, x, ws, wr, b2d)


_MESH = plsc.VectorSubcoreMesh(
    core_axis_name="c", subcore_axis_name="s", num_cores=NC, num_subcores=NS
)


@functools.partial(
    pl.kernel,
    mesh=_MESH,
    out_type=jax.ShapeDtypeStruct((N_EDGES, D_OUT), jnp.float32),
    scratch_types=[
        pltpu.VMEM((2, 2, CHUNK), jnp.int32),
        pltpu.VMEM((2, CHUNK, D_OUT), jnp.float32),
        pltpu.VMEM((2, CHUNK, D_OUT), jnp.float32),
        pltpu.VMEM((2, CHUNK, D_OUT), jnp.float32),
        [pltpu.SemaphoreType.DMA] * 2,
        [pltpu.SemaphoreType.DMA] * 2,
        [pltpu.SemaphoreType.DMA] * 2,
        [pltpu.SemaphoreType.DMA] * 2,
        [pltpu.SemaphoreType.DMA] * 2,
    ],
)
def _sc_gather_sum(ts_hbm, tr_hbm, ep_hbm, sidx_hbm, ridx_hbm, out_hbm,
                   idx_v, a_v, b_v, o_v, si, sa, sb, sc, so):
    wid = lax.axis_index("s") * NC + lax.axis_index("c")
    base = wid * PER_W

    def issue_gathers(ci, k):
        off = base + ci * CHUNK
        pltpu.async_copy(ts_hbm.at[idx_v.at[k, 0]], a_v.at[k], sa[k])
        pltpu.async_copy(tr_hbm.at[idx_v.at[k, 1]], b_v.at[k], sb[k])
        pltpu.async_copy(ep_hbm.at[pl.ds(off, CHUNK)], o_v.at[k], sc[k])

    def wait_gathers(k):
        pltpu.make_async_copy(ts_hbm.at[idx_v.at[k, 0]], a_v.at[k], sa[k]).wait()
        pltpu.make_async_copy(tr_hbm.at[idx_v.at[k, 1]], b_v.at[k], sb[k]).wait()
        pltpu.make_async_copy(ep_hbm.at[pl.ds(0, CHUNK)], o_v.at[k], sc[k]).wait()

    def prefetch_idx(ci, k):
        off = base + ci * CHUNK
        pltpu.async_copy(sidx_hbm.at[pl.ds(off, CHUNK)], idx_v.at[k, 0], si[k])
        pltpu.async_copy(ridx_hbm.at[pl.ds(off, CHUNK)], idx_v.at[k, 1], si[k])

    def wait_idx(k):
        pltpu.make_async_copy(sidx_hbm.at[pl.ds(0, CHUNK)], idx_v.at[k, 0], si[k]).wait()
        pltpu.make_async_copy(ridx_hbm.at[pl.ds(0, CHUNK)], idx_v.at[k, 1], si[k]).wait()

    def drain_out(k):
        pltpu.make_async_copy(o_v.at[k], out_hbm.at[pl.ds(0, CHUNK)], so[k]).wait()

    pltpu.sync_copy(sidx_hbm.at[pl.ds(base, CHUNK)], idx_v.at[0, 0])
    pltpu.sync_copy(ridx_hbm.at[pl.ds(base, CHUNK)], idx_v.at[0, 1])
    issue_gathers(0, 0)
    prefetch_idx(1, 1)

    def chunk_body(ci, k):
            nk = 1 - k

            @pl.when(ci + 1 < N_CHUNKS)
            def _():
                wait_idx(nk)

                @pl.when(ci >= 1)
                def _():
                    drain_out(nk)

                issue_gathers(ci + 1, nk)

            wait_gathers(k)

            @pl.when(ci + 2 < N_CHUNKS)
            def _():
                prefetch_idx(ci + 2, k)

            @plsc.parallel_loop(0, CHUNK, unroll=2)
            def _(e):
                for j in range(D_OUT // 16):
                    sl = pl.ds(j * 16, 16)
                    plsc.addupdate(o_v.at[k, e, sl], a_v[k, e, sl] + b_v[k, e, sl])
            off = base + ci * CHUNK
            pltpu.async_copy(o_v.at[k], out_hbm.at[pl.ds(off, CHUNK)], so[k])

    def pair_body(g, carry):
        chunk_body(g, 0)
        chunk_body(g + 1, 1)
        return carry

    lax.fori_loop(0, N_CHUNKS // 2, lambda g, c: pair_body(g * 2, c), 0)
    if N_CHUNKS % 2:
        chunk_body(N_CHUNKS - 1, 0)
    drain_out((N_CHUNKS - 2) % 2)
    drain_out((N_CHUNKS - 1) % 2)


def kernel(x, edge_index, edge_attr, coords, W, b):
    ws = W[:D_FEAT]
    wr = W[D_FEAT:2 * D_FEAT]
    wep = W[2 * D_FEAT:]
    b2d = b.reshape(1, D_OUT)

    ep, ts, tr = _tc_stage(edge_attr, wep, x, ws, wr, b2d)

    sidx = edge_index[0].astype(jnp.int32)
    ridx = edge_index[1].astype(jnp.int32)
    edge_attr_ = _sc_gather_sum(ts, tr, ep, sidx, ridx)
    return (coords, x, edge_attr_, edge_index)
